# baseline (speedup 1.0000x reference)
import functools

import jax
import jax.numpy as jnp
from jax import lax
from jax.experimental import pallas as pl
from jax.experimental.pallas import tpu as pltpu
from jax.experimental.pallas import tpu_sc as plsc

N_NODES = 10000
NPAD = 10240
N_EDGES = 320000
D = 64
NC = 2
NS = 16
EB = 80
G = 5
EPAD = 320000
EROWS = EPAD // EB
ROWS_PER_TILE = EROWS // NS
NODES_PER_TILE = NPAD // NS
NCHUNK = 128
TAIL = N_NODES % NCHUNK
NGROUPS = ROWS_PER_TILE // G

_mesh = plsc.VectorSubcoreMesh(core_axis_name="c", subcore_axis_name="s")


@functools.partial(
    pl.kernel,
    mesh=_mesh,
    compiler_params=pltpu.CompilerParams(use_tc_tiling_on_sc=False),
    out_type=(
        jax.ShapeDtypeStruct((N_NODES, 3 * NC * D), jnp.float32),
        jax.ShapeDtypeStruct((NC * NPAD, D), jnp.float32),
    ),
    scratch_types=[
        pltpu.VMEM((4, G, EB), jnp.int32),
        pltpu.VMEM((4, G, EB), jnp.int32),
        pltpu.VMEM((3, G, EB, D), jnp.float32),
        pltpu.VMEM((EB,), jnp.float32),
        pltpu.VMEM((NCHUNK, D), jnp.float32),
        pltpu.VMEM((NCHUNK,), jnp.float32),
        pltpu.VMEM((NODES_PER_TILE,), jnp.float32),
        pltpu.VMEM_SHARED((NPAD, D), jnp.float32),
        pltpu.VMEM_SHARED((NPAD,), jnp.float32),
        pltpu.SemaphoreType.DMA,
        pltpu.SemaphoreType.DMA,
        pltpu.SemaphoreType.DMA,
        pltpu.SemaphoreType.DMA,
    ],
)
def _gin_sc(x_hbm, srcs_hbm, dst_hbm, ones_hbm, zeros_hbm, zdeg_hbm,
            out_hbm, h1_hbm,
            src_v, dst_v, rows_v, ones_v, hbuf_v, deg_v, rinv_v,
            acc_s, deg_s, sem_i, sem_g, sem_s, sem_d):
    c = lax.axis_index("c")
    s = lax.axis_index("s")
    node_base = s * NODES_PER_TILE
    row_base = s * ROWS_PER_TILE
    coff = c * NPAD
    col = c * D

    pltpu.sync_copy(ones_hbm, ones_v)
    pltpu.sync_copy(zeros_hbm, acc_s.at[pl.ds(node_base, NODES_PER_TILE)])
    pltpu.sync_copy(zdeg_hbm, deg_s.at[pl.ds(node_base, NODES_PER_TILE)])
    plsc.subcore_barrier()

    def _idx_load(g):
        off = row_base + g * G
        st = g % 4
        pltpu.async_copy(srcs_hbm.at[c].at[pl.ds(off, G)], src_v.at[st], sem_i)
        pltpu.async_copy(dst_hbm.at[pl.ds(off, G)], dst_v.at[st], sem_i)

    def _idx_wait(g):
        off = row_base + g * G
        st = g % 4
        pltpu.make_async_copy(
            srcs_hbm.at[c].at[pl.ds(off, G)], src_v.at[st], sem_i).wait()
        pltpu.make_async_copy(
            dst_hbm.at[pl.ds(off, G)], dst_v.at[st], sem_i).wait()

    def _edge_pass(tbl, with_deg):
        _idx_load(0)
        _idx_load(1)
        _idx_wait(0)
        for b in range(G):
            pltpu.async_copy(tbl.at[src_v.at[0, b]], rows_v.at[0, b], sem_g)

        def _group(g, carry):
            r_cur = g % 3
            r_nxt = (g + 1) % 3
            i_cur = g % 4
            i_nxt = (g + 1) % 4
            i_old = (g + 2) % 4

            @pl.when(g > 1)
            def _():
                for b in range(G):
                    pltpu.make_async_copy(
                        rows_v.at[r_nxt, b], acc_s.at[dst_v.at[i_old, b]],
                        sem_s).wait()

            @pl.when(g + 2 < NGROUPS)
            def _():
                _idx_load(g + 2)

            @pl.when(g + 1 < NGROUPS)
            def _():
                _idx_wait(g + 1)
                for b in range(G):
                    pltpu.async_copy(
                        tbl.at[src_v.at[i_nxt, b]], rows_v.at[r_nxt, b], sem_g)

            for b in range(G):
                pltpu.make_async_copy(
                    tbl.at[src_v.at[i_cur, b]], rows_v.at[r_cur, b], sem_g).wait()
                pltpu.async_copy(
                    rows_v.at[r_cur, b], acc_s.at[dst_v.at[i_cur, b]],
                    sem_s, add=True)
                if with_deg:
                    pltpu.async_copy(
                        ones_v, deg_s.at[dst_v.at[i_cur, b]], sem_d, add=True)

            return carry

        lax.fori_loop(0, NGROUPS, _group, 0)

        for gl in (NGROUPS - 2, NGROUPS - 1):
            for b in range(G):
                pltpu.make_async_copy(
                    rows_v.at[gl % 3, b], acc_s.at[dst_v.at[gl % 4, b]],
                    sem_s).wait()
        if with_deg:
            def _deg_drain(j, carry):
                pltpu.make_async_copy(
                    ones_v, deg_s.at[dst_v.at[0, 0]], sem_d).wait()
                return carry

            lax.fori_loop(0, NGROUPS * G, _deg_drain, 0)

    _edge_pass(x_hbm, True)
    plsc.subcore_barrier()

    def _out_write(base, ocol):
        full = base + NCHUNK <= N_NODES
        part = jnp.logical_and(base < N_NODES, jnp.logical_not(full))

        @pl.when(full)
        def _():
            pltpu.sync_copy(
                hbuf_v, out_hbm.at[pl.ds(base, NCHUNK), pl.ds(ocol, D)])

        @pl.when(part)
        def _():
            pltpu.sync_copy(
                hbuf_v.at[pl.ds(0, TAIL)],
                out_hbm.at[pl.ds(base, TAIL), pl.ds(ocol, D)])

    def _finish(col_base, table, first):
        for k in range(NODES_PER_TILE // NCHUNK):
            base = node_base + k * NCHUNK
            pltpu.sync_copy(acc_s.at[pl.ds(base, NCHUNK)], hbuf_v)
            if first:
                pltpu.sync_copy(deg_s.at[pl.ds(base, NCHUNK)], deg_v)

            def _div(grp, carry):
                if first:
                    dvec = deg_v[pl.ds(grp * 16, 16)]
                    rinv = 1.0 / jnp.maximum(dvec, 1.0)
                    rinv_v[pl.ds(k * NCHUNK + grp * 16, 16)] = rinv
                else:
                    rinv = rinv_v[pl.ds(k * NCHUNK + grp * 16, 16)]
                for kk in range(16):
                    i = grp * 16 + kk
                    rv = jnp.full((16,), rinv[kk], jnp.float32)
                    for q in range(D // 16):
                        sl = pl.ds(q * 16, 16)
                        hbuf_v[i, sl] = hbuf_v[i, sl] * rv
                return carry

            lax.fori_loop(0, NCHUNK // 16, _div, 0)
            if table is not None:
                pltpu.sync_copy(hbuf_v, table.at[pl.ds(coff + base, NCHUNK)])
            _out_write(base, col_base + col)

    _finish(D * NC, h1_hbm, True)

    for k in range(NODES_PER_TILE // NCHUNK):
        base = node_base + k * NCHUNK
        pltpu.sync_copy(x_hbm.at[pl.ds(coff + base, NCHUNK)], hbuf_v)
        _out_write(base, col)

    pltpu.sync_copy(zeros_hbm, acc_s.at[pl.ds(node_base, NODES_PER_TILE)])
    plsc.subcore_barrier()

    _edge_pass(h1_hbm, False)
    plsc.subcore_barrier()

    _finish(2 * D * NC, None, False)


def kernel(x, edge_index):
    epad = EPAD - N_EDGES
    src = jnp.concatenate(
        [edge_index[0], jnp.zeros((epad,), jnp.int32)]).reshape(EROWS, EB)
    pad_dst = N_NODES + jnp.arange(epad, dtype=jnp.int32) % (NPAD - N_NODES)
    dst = jnp.concatenate([edge_index[1], pad_dst]).reshape(EROWS, EB)
    srcs = jnp.stack([src, src + NPAD])
    pad = ((0, NPAD - N_NODES), (0, 0))
    x_flat = jnp.concatenate(
        [jnp.pad(x[:, :D], pad), jnp.pad(x[:, D:], pad)], axis=0)
    ones = jnp.ones((EB,), jnp.float32)
    zeros64 = jnp.zeros((NODES_PER_TILE, D), jnp.float32)
    zdeg = jnp.zeros((NODES_PER_TILE,), jnp.float32)
    out, _ = _gin_sc(x_flat, srcs, dst, ones, zeros64, zdeg)
    return out


# speedup vs baseline: 1.0742x; 1.0742x over previous
import functools

import jax
import jax.numpy as jnp
from jax import lax
from jax.experimental import pallas as pl
from jax.experimental.pallas import tpu as pltpu
from jax.experimental.pallas import tpu_sc as plsc

N_NODES = 10000
NPAD = 10240
N_EDGES = 320000
D = 64
NC = 2
NS = 16
EB = 80
G = 5
EPAD = 320000
EROWS = EPAD // EB
ROWS_PER_TILE = EROWS // NS
NODES_PER_TILE = NPAD // NS
NCHUNK = 128
TAIL = N_NODES % NCHUNK
NGROUPS = ROWS_PER_TILE // G

_mesh = plsc.VectorSubcoreMesh(core_axis_name="c", subcore_axis_name="s")


@functools.partial(
    pl.kernel,
    mesh=_mesh,
    compiler_params=pltpu.CompilerParams(use_tc_tiling_on_sc=False),
    out_type=(
        jax.ShapeDtypeStruct((N_NODES, 3 * NC * D), jnp.float32),
        jax.ShapeDtypeStruct((NC * NPAD, D), jnp.float32),
        jax.ShapeDtypeStruct((NC * NPAD, D), jnp.float32),
    ),
    scratch_types=[
        pltpu.VMEM((3, G, EB), jnp.int32),
        pltpu.VMEM((3, G, EB), jnp.int32),
        pltpu.VMEM((2, G, EB, D), jnp.float32),
        pltpu.VMEM((EB,), jnp.float32),
        pltpu.VMEM((NCHUNK, D), jnp.float32),
        pltpu.VMEM((NCHUNK,), jnp.float32),
        pltpu.VMEM((NODES_PER_TILE,), jnp.float32),
        pltpu.VMEM_SHARED((NPAD, D), jnp.float32),
        pltpu.VMEM_SHARED((NPAD,), jnp.float32),
        pltpu.SemaphoreType.DMA,
        pltpu.SemaphoreType.DMA,
        pltpu.SemaphoreType.DMA,
        pltpu.SemaphoreType.DMA,
    ],
)
def _gin_sc(x_hbm, src_hbm, dst_hbm, ones_hbm, zeros_hbm, zdeg_hbm,
            out_hbm, h1_hbm, xf_hbm,
            src_v, dst_v, rows_v, ones_v, hbuf_v, deg_v, rinv_v,
            acc_s, deg_s, sem_i, sem_g, sem_s, sem_d):
    c = lax.axis_index("c")
    s = lax.axis_index("s")
    node_base = s * NODES_PER_TILE
    row_base = s * ROWS_PER_TILE
    coff = c * NPAD
    col = c * D

    pltpu.sync_copy(ones_hbm, ones_v)
    pltpu.sync_copy(zeros_hbm, acc_s.at[pl.ds(node_base, NODES_PER_TILE)])
    pltpu.sync_copy(zdeg_hbm, deg_s.at[pl.ds(node_base, NODES_PER_TILE)])

    for k in range(NODES_PER_TILE // NCHUNK):
        base = node_base + k * NCHUNK
        full = base + NCHUNK <= N_NODES
        part = jnp.logical_and(base < N_NODES, jnp.logical_not(full))

        @pl.when(full)
        def _():
            pltpu.sync_copy(
                x_hbm.at[pl.ds(base, NCHUNK), pl.ds(col, D)], hbuf_v)
            pltpu.sync_copy(hbuf_v, xf_hbm.at[pl.ds(coff + base, NCHUNK)])
            pltpu.sync_copy(
                hbuf_v, out_hbm.at[pl.ds(base, NCHUNK), pl.ds(col, D)])

        @pl.when(part)
        def _():
            pltpu.sync_copy(
                x_hbm.at[pl.ds(base, TAIL), pl.ds(col, D)],
                hbuf_v.at[pl.ds(0, TAIL)])
            pltpu.sync_copy(
                hbuf_v.at[pl.ds(0, TAIL)], xf_hbm.at[pl.ds(coff + base, TAIL)])
            pltpu.sync_copy(
                hbuf_v.at[pl.ds(0, TAIL)],
                out_hbm.at[pl.ds(base, TAIL), pl.ds(col, D)])

    plsc.subcore_barrier()

    def _idx_load(g):
        off = row_base + g * G
        st = g % 3
        pltpu.async_copy(src_hbm.at[pl.ds(off, G)], src_v.at[st], sem_i)
        pltpu.async_copy(dst_hbm.at[pl.ds(off, G)], dst_v.at[st], sem_i)

    def _idx_wait(g):
        off = row_base + g * G
        st = g % 3
        pltpu.make_async_copy(
            src_hbm.at[pl.ds(off, G)], src_v.at[st], sem_i).wait()
        pltpu.make_async_copy(
            dst_hbm.at[pl.ds(off, G)], dst_v.at[st], sem_i).wait()
        for b in range(G):
            for q in range(EB // 16):
                sl = pl.ds(q * 16, 16)
                src_v[st, b, sl] = src_v[st, b, sl] + coff

    def _edge_pass(tbl, with_deg):
        _idx_load(0)
        _idx_load(1)
        _idx_wait(0)
        for b in range(G):
            pltpu.async_copy(tbl.at[src_v.at[0, b]], rows_v.at[0, b], sem_g)

        def _group(g, carry):
            r_cur = g % 2
            r_nxt = 1 - r_cur
            i_cur = g % 3
            i_nxt = (g + 1) % 3
            i_old = (g + 2) % 3

            @pl.when(g > 0)
            def _():
                for b in range(G):
                    pltpu.make_async_copy(
                        rows_v.at[r_nxt, b], acc_s.at[dst_v.at[i_old, b]],
                        sem_s).wait()

            @pl.when(g + 2 < NGROUPS)
            def _():
                _idx_load(g + 2)

            @pl.when(g + 1 < NGROUPS)
            def _():
                _idx_wait(g + 1)
                for b in range(G):
                    pltpu.async_copy(
                        tbl.at[src_v.at[i_nxt, b]], rows_v.at[r_nxt, b], sem_g)

            for b in range(G):
                pltpu.make_async_copy(
                    tbl.at[src_v.at[i_cur, b]], rows_v.at[r_cur, b], sem_g).wait()
                pltpu.async_copy(
                    rows_v.at[r_cur, b], acc_s.at[dst_v.at[i_cur, b]],
                    sem_s, add=True)
                if with_deg:
                    pltpu.async_copy(
                        ones_v, deg_s.at[dst_v.at[i_cur, b]], sem_d, add=True)

            return carry

        lax.fori_loop(0, NGROUPS, _group, 0)

        for b in range(G):
            pltpu.make_async_copy(
                rows_v.at[(NGROUPS - 1) % 2, b],
                acc_s.at[dst_v.at[(NGROUPS - 1) % 3, b]], sem_s).wait()
        if with_deg:
            def _deg_drain(j, carry):
                pltpu.make_async_copy(
                    ones_v, deg_s.at[dst_v.at[0, 0]], sem_d).wait()
                return carry

            lax.fori_loop(0, NGROUPS * G, _deg_drain, 0)

    _edge_pass(xf_hbm, True)
    plsc.subcore_barrier()

    def _out_write(base, ocol):
        full = base + NCHUNK <= N_NODES
        part = jnp.logical_and(base < N_NODES, jnp.logical_not(full))

        @pl.when(full)
        def _():
            pltpu.sync_copy(
                hbuf_v, out_hbm.at[pl.ds(base, NCHUNK), pl.ds(ocol, D)])

        @pl.when(part)
        def _():
            pltpu.sync_copy(
                hbuf_v.at[pl.ds(0, TAIL)],
                out_hbm.at[pl.ds(base, TAIL), pl.ds(ocol, D)])

    def _finish(col_base, table, first):
        for k in range(NODES_PER_TILE // NCHUNK):
            base = node_base + k * NCHUNK
            pltpu.sync_copy(acc_s.at[pl.ds(base, NCHUNK)], hbuf_v)
            if first:
                pltpu.sync_copy(deg_s.at[pl.ds(base, NCHUNK)], deg_v)

            def _div(grp, carry):
                if first:
                    dvec = deg_v[pl.ds(grp * 16, 16)]
                    rinv = 1.0 / jnp.maximum(dvec, 1.0)
                    rinv_v[pl.ds(k * NCHUNK + grp * 16, 16)] = rinv
                else:
                    rinv = rinv_v[pl.ds(k * NCHUNK + grp * 16, 16)]
                for kk in range(16):
                    i = grp * 16 + kk
                    rv = jnp.full((16,), rinv[kk], jnp.float32)
                    for q in range(D // 16):
                        sl = pl.ds(q * 16, 16)
                        hbuf_v[i, sl] = hbuf_v[i, sl] * rv
                return carry

            lax.fori_loop(0, NCHUNK // 16, _div, 0)
            if table is not None:
                pltpu.sync_copy(hbuf_v, table.at[pl.ds(coff + base, NCHUNK)])
            _out_write(base, col_base + col)

    _finish(D * NC, h1_hbm, True)

    pltpu.sync_copy(zeros_hbm, acc_s.at[pl.ds(node_base, NODES_PER_TILE)])
    plsc.subcore_barrier()

    _edge_pass(h1_hbm, False)
    plsc.subcore_barrier()

    _finish(2 * D * NC, None, False)


def kernel(x, edge_index):
    src = edge_index[0].reshape(EROWS, EB)
    dst = edge_index[1].reshape(EROWS, EB)
    ones = jnp.ones((EB,), jnp.float32)
    zeros64 = jnp.zeros((NODES_PER_TILE, D), jnp.float32)
    zdeg = jnp.zeros((NODES_PER_TILE,), jnp.float32)
    out, _, _ = _gin_sc(x, src, dst, ones, zeros64, zdeg)
    return out


# speedup vs baseline: 1.0868x; 1.0117x over previous
import functools

import jax
import jax.numpy as jnp
from jax import lax
from jax.experimental import pallas as pl
from jax.experimental.pallas import tpu as pltpu
from jax.experimental.pallas import tpu_sc as plsc

N_NODES = 10000
NPAD = 10240
N_EDGES = 320000
D = 64
NC = 2
NS = 16
EB = 80
G = 5
EPAD = 320000
EROWS = EPAD // EB
ROWS_PER_TILE = EROWS // NS
NODES_PER_TILE = NPAD // NS
NCHUNK = 128
TAIL = N_NODES % NCHUNK
NGROUPS = ROWS_PER_TILE // G

_mesh = plsc.VectorSubcoreMesh(core_axis_name="c", subcore_axis_name="s")


@functools.partial(
    pl.kernel,
    mesh=_mesh,
    compiler_params=pltpu.CompilerParams(use_tc_tiling_on_sc=False),
    out_type=(
        jax.ShapeDtypeStruct((N_NODES, 3 * NC * D), jnp.float32),
        jax.ShapeDtypeStruct((NC * NPAD, D), jnp.float32),
        jax.ShapeDtypeStruct((NC * NPAD, D), jnp.float32),
    ),
    scratch_types=[
        pltpu.VMEM((3, G, EB), jnp.int32),
        pltpu.VMEM((3, G, EB), jnp.int32),
        pltpu.VMEM((2, G, EB, D), jnp.float32),
        pltpu.VMEM((EB,), jnp.float32),
        pltpu.VMEM((2, NCHUNK, D), jnp.float32),
        pltpu.VMEM((NCHUNK,), jnp.float32),
        pltpu.VMEM((NODES_PER_TILE,), jnp.float32),
        pltpu.VMEM_SHARED((NPAD, D), jnp.float32),
        pltpu.VMEM_SHARED((NPAD,), jnp.float32),
        pltpu.SemaphoreType.DMA,
        pltpu.SemaphoreType.DMA,
        pltpu.SemaphoreType.DMA,
        pltpu.SemaphoreType.DMA,
    ],
)
def _gin_sc(x_hbm, src_hbm, dst_hbm, ones_hbm, zeros_hbm, zdeg_hbm,
            out_hbm, h1_hbm, xf_hbm,
            src_v, dst_v, rows_v, ones_v, hbuf_v, deg_v, rinv_v,
            acc_s, deg_s, sem_i, sem_g, sem_s, sem_d):
    c = lax.axis_index("c")
    s = lax.axis_index("s")
    node_base = s * NODES_PER_TILE
    row_base = s * ROWS_PER_TILE
    coff = c * NPAD
    col = c * D

    pltpu.sync_copy(ones_hbm, ones_v)
    pltpu.sync_copy(zeros_hbm, acc_s.at[pl.ds(node_base, NODES_PER_TILE)])
    pltpu.sync_copy(zdeg_hbm, deg_s.at[pl.ds(node_base, NODES_PER_TILE)])

    for k in range(NODES_PER_TILE // NCHUNK):
        base = node_base + k * NCHUNK
        full = base + NCHUNK <= N_NODES
        part = jnp.logical_and(base < N_NODES, jnp.logical_not(full))

        hb = hbuf_v.at[k % 2]

        @pl.when(full)
        def _():
            pltpu.sync_copy(
                x_hbm.at[pl.ds(base, NCHUNK), pl.ds(col, D)], hb)
            pltpu.sync_copy(hb, xf_hbm.at[pl.ds(coff + base, NCHUNK)])
            pltpu.sync_copy(
                hb, out_hbm.at[pl.ds(base, NCHUNK), pl.ds(col, D)])

        @pl.when(part)
        def _():
            pltpu.sync_copy(
                x_hbm.at[pl.ds(base, TAIL), pl.ds(col, D)],
                hb.at[pl.ds(0, TAIL)])
            pltpu.sync_copy(
                hb.at[pl.ds(0, TAIL)], xf_hbm.at[pl.ds(coff + base, TAIL)])
            pltpu.sync_copy(
                hb.at[pl.ds(0, TAIL)],
                out_hbm.at[pl.ds(base, TAIL), pl.ds(col, D)])

    plsc.subcore_barrier()

    def _idx_load(g):
        off = row_base + g * G
        st = g % 3
        pltpu.async_copy(src_hbm.at[pl.ds(off, G)], src_v.at[st], sem_i)
        pltpu.async_copy(dst_hbm.at[pl.ds(off, G)], dst_v.at[st], sem_i)

    def _idx_wait(g):
        off = row_base + g * G
        st = g % 3
        pltpu.make_async_copy(
            src_hbm.at[pl.ds(off, G)], src_v.at[st], sem_i).wait()
        pltpu.make_async_copy(
            dst_hbm.at[pl.ds(off, G)], dst_v.at[st], sem_i).wait()
        for b in range(G):
            for q in range(EB // 16):
                sl = pl.ds(q * 16, 16)
                src_v[st, b, sl] = src_v[st, b, sl] + coff

    def _edge_pass(tbl, with_deg):
        _idx_load(0)
        _idx_load(1)
        _idx_wait(0)
        for b in range(G):
            pltpu.async_copy(tbl.at[src_v.at[0, b]], rows_v.at[0, b], sem_g)

        def _group(g, carry):
            r_cur = g % 2
            r_nxt = 1 - r_cur
            i_cur = g % 3
            i_nxt = (g + 1) % 3
            i_old = (g + 2) % 3

            @pl.when(g > 0)
            def _():
                for b in range(G):
                    pltpu.make_async_copy(
                        rows_v.at[r_nxt, b], acc_s.at[dst_v.at[i_old, b]],
                        sem_s).wait()

            @pl.when(g + 2 < NGROUPS)
            def _():
                _idx_load(g + 2)

            @pl.when(g + 1 < NGROUPS)
            def _():
                _idx_wait(g + 1)
                for b in range(G):
                    pltpu.async_copy(
                        tbl.at[src_v.at[i_nxt, b]], rows_v.at[r_nxt, b], sem_g)

            for b in range(G):
                pltpu.make_async_copy(
                    tbl.at[src_v.at[i_cur, b]], rows_v.at[r_cur, b], sem_g).wait()
                pltpu.async_copy(
                    rows_v.at[r_cur, b], acc_s.at[dst_v.at[i_cur, b]],
                    sem_s, add=True)
                if with_deg:
                    pltpu.async_copy(
                        ones_v, deg_s.at[dst_v.at[i_cur, b]], sem_d, add=True)

            return carry

        lax.fori_loop(0, NGROUPS, _group, 0)

        for b in range(G):
            pltpu.make_async_copy(
                rows_v.at[(NGROUPS - 1) % 2, b],
                acc_s.at[dst_v.at[(NGROUPS - 1) % 3, b]], sem_s).wait()
        if with_deg:
            def _deg_drain(j, carry):
                pltpu.make_async_copy(
                    ones_v, deg_s.at[dst_v.at[0, 0]], sem_d).wait()
                return carry

            lax.fori_loop(0, NGROUPS * G, _deg_drain, 0)

    _edge_pass(xf_hbm, True)
    plsc.subcore_barrier()

    def _finish(col_base, table, first):
        nk = NODES_PER_TILE // NCHUNK

        def _wcond(k):
            base = node_base + k * NCHUNK
            full = base + NCHUNK <= N_NODES
            part = jnp.logical_and(base < N_NODES, jnp.logical_not(full))
            return base, full, part

        def _drain(k):
            base, full, part = _wcond(k)
            hb = hbuf_v.at[k % 2]
            if table is not None:
                pltpu.make_async_copy(
                    hb, table.at[pl.ds(coff + base, NCHUNK)], sem_s).wait()
            ocol = col_base + col

            @pl.when(full)
            def _():
                pltpu.make_async_copy(
                    hb, out_hbm.at[pl.ds(base, NCHUNK), pl.ds(ocol, D)],
                    sem_s).wait()

            @pl.when(part)
            def _():
                pltpu.make_async_copy(
                    hb.at[pl.ds(0, TAIL)],
                    out_hbm.at[pl.ds(base, TAIL), pl.ds(ocol, D)],
                    sem_s).wait()

        for k in range(nk):
            hb = hbuf_v.at[k % 2]
            base, full, part = _wcond(k)
            if k >= 2:
                _drain(k - 2)
            pltpu.sync_copy(acc_s.at[pl.ds(base, NCHUNK)], hb)
            if first:
                pltpu.sync_copy(deg_s.at[pl.ds(base, NCHUNK)], deg_v)

            def _div(grp, carry):
                if first:
                    dvec = deg_v[pl.ds(grp * 16, 16)]
                    rinv = 1.0 / jnp.maximum(dvec, 1.0)
                    rinv_v[pl.ds(k * NCHUNK + grp * 16, 16)] = rinv
                else:
                    rinv = rinv_v[pl.ds(k * NCHUNK + grp * 16, 16)]
                for kk in range(16):
                    i = grp * 16 + kk
                    rv = jnp.full((16,), rinv[kk], jnp.float32)
                    for q in range(D // 16):
                        sl = pl.ds(q * 16, 16)
                        hb[i, sl] = hb[i, sl] * rv
                return carry

            lax.fori_loop(0, NCHUNK // 16, _div, 0)
            if table is not None:
                pltpu.async_copy(
                    hb, table.at[pl.ds(coff + base, NCHUNK)], sem_s)
            ocol = col_base + col

            @pl.when(full)
            def _():
                pltpu.async_copy(
                    hb, out_hbm.at[pl.ds(base, NCHUNK), pl.ds(ocol, D)], sem_s)

            @pl.when(part)
            def _():
                pltpu.async_copy(
                    hb.at[pl.ds(0, TAIL)],
                    out_hbm.at[pl.ds(base, TAIL), pl.ds(ocol, D)], sem_s)

        _drain(nk - 2)
        _drain(nk - 1)

    _finish(D * NC, h1_hbm, True)

    pltpu.sync_copy(zeros_hbm, acc_s.at[pl.ds(node_base, NODES_PER_TILE)])
    plsc.subcore_barrier()

    _edge_pass(h1_hbm, False)
    plsc.subcore_barrier()

    _finish(2 * D * NC, None, False)


def kernel(x, edge_index):
    src = edge_index[0].reshape(EROWS, EB)
    dst = edge_index[1].reshape(EROWS, EB)
    ones = jnp.ones((EB,), jnp.float32)
    zeros64 = jnp.zeros((NODES_PER_TILE, D), jnp.float32)
    zdeg = jnp.zeros((NODES_PER_TILE,), jnp.float32)
    out, _, _ = _gin_sc(x, src, dst, ones, zeros64, zdeg)
    return out
